# Initial kernel scaffold; baseline (speedup 1.0000x reference)
import functools

import jax
import jax.numpy as jnp
from jax import lax
from jax.experimental import pallas as pl
from jax.experimental.pallas import tpu as pltpu
from jax.experimental.pallas import tpu_sc as plsc

_N, _T, _C = 4096, 200, 64
_ALPHA = 0.5
_L = 16
_NC, _NS = 2, 16
_NW = _NC * _NS
_EPW = _N * _T // _NW
_K = 128
_NCHUNK = _EPW // _K
_VPC = _K // _L


def _sc_body(logp_hbm, z_hbm, y_hbm, out_hbm, z_v, y_v, idx_v, gbuf, acc_v,
             sem0, sem1):
    wid = lax.axis_index("s") * _NC + lax.axis_index("c")
    base_e = wid * _EPW

    pltpu.sync_copy(z_hbm.at[pl.ds(base_e, _EPW)], z_v)
    pltpu.sync_copy(y_hbm.at[pl.ds(base_e, _EPW)], y_v)

    iota = lax.iota(jnp.int32, _L)
    sems = (sem0, sem1)

    def flat_idx(off):
        e = off + iota
        z = z_v[pl.ds(off, _L)]
        y = y_v[pl.ds(off, _L)]
        n_loc = e // _T
        t = e - n_loc * _T
        tt = jnp.minimum(t + z, _T - 1)
        flat = (wid * (_EPW // _T) + n_loc) * (_T * _C) + tt * _C + y
        return flat, t, z

    def fire(j, b):
        for v in range(_VPC):
            off = j * _K + v * _L
            flat, _, _ = flat_idx(off)
            idx_v[b, pl.ds(v * _L, _L)] = flat // _L
        pltpu.make_async_copy(logp_hbm.at[idx_v.at[b]], gbuf.at[b],
                              sems[b]).start()

    def consume(j, b, acc):
        pltpu.make_async_copy(logp_hbm.at[idx_v.at[b]], gbuf.at[b],
                              sems[b]).wait()
        for v in range(_VPC):
            off = j * _K + v * _L
            flat, t, z = flat_idx(off)
            lane = flat - (flat // _L) * _L
            row = v * _L + iota
            lc = plsc.load_gather(gbuf.at[b], [row, lane])
            tf = t.astype(jnp.float32)
            zf = z.astype(jnp.float32)
            w = (1.0 - tf * (1.0 / _T)) * (1.0 - zf * (1.0 / _T))
            acc = acc + (-_ALPHA) * lc - (1.0 - _ALPHA) * jnp.exp(lc) * w
        return acc

    fire(0, 0)
    fire(1, 1)

    def body(j2, acc):
        j = j2 * 2
        acc = consume(j, 0, acc)
        fire(j + 2, 0)
        acc = consume(j + 1, 1, acc)
        fire(j + 3, 1)
        return acc

    acc = jnp.zeros((_L,), jnp.float32)
    acc = lax.fori_loop(0, _NCHUNK // 2 - 1, body, acc)
    acc = consume(_NCHUNK - 2, 0, acc)
    acc = consume(_NCHUNK - 1, 1, acc)

    acc_v[...] = acc
    pltpu.sync_copy(acc_v, out_hbm.at[wid])


_sc_kernel = functools.partial(
    pl.kernel,
    mesh=plsc.VectorSubcoreMesh(core_axis_name="c", subcore_axis_name="s"),
    out_type=jax.ShapeDtypeStruct((_NW, _L), jnp.float32),
    scratch_types=[
        pltpu.VMEM((_EPW,), jnp.int32),
        pltpu.VMEM((_EPW,), jnp.int32),
        pltpu.VMEM((2, _K), jnp.int32),
        pltpu.VMEM((2, _K, _L), jnp.float32),
        pltpu.VMEM((_L,), jnp.float32),
        pltpu.SemaphoreType.DMA,
        pltpu.SemaphoreType.DMA,
    ],
)(_sc_kernel_body := None) if False else None


def _make_sc_kernel():
    return pl.kernel(
        _sc_body,
        mesh=plsc.VectorSubcoreMesh(core_axis_name="c", subcore_axis_name="s"),
        out_type=jax.ShapeDtypeStruct((_NW, _L), jnp.float32),
        scratch_types=[
            pltpu.VMEM((_EPW,), jnp.int32),
            pltpu.VMEM((_EPW,), jnp.int32),
            pltpu.VMEM((2, _K), jnp.int32),
            pltpu.VMEM((2, _K, _L), jnp.float32),
            pltpu.VMEM((_L,), jnp.float32),
            pltpu.SemaphoreType.DMA,
            pltpu.SemaphoreType.DMA,
        ],
    )


_sc_kernel = _make_sc_kernel()


def kernel(log_class_probabilities, timestamps_left, y_true):
    logp2 = log_class_probabilities.reshape(_N * _T * _C // _L, _L)
    z = timestamps_left.astype(jnp.int32).reshape(-1)
    y = y_true.astype(jnp.int32).reshape(-1)
    partials = _sc_kernel(logp2, z, y)
    return partials.sum() * (1.0 / _N)


# speedup vs baseline: 2.6959x; 2.6959x over previous
import jax
import jax.numpy as jnp
from jax import lax
from jax.experimental import pallas as pl
from jax.experimental.pallas import tpu as pltpu
from jax.experimental.pallas import tpu_sc as plsc

_N, _T, _C = 4096, 200, 64
_ALPHA = 0.5
_L = 16
_NC, _NS = 2, 16
_NW = _NC * _NS
_EPW = _N * _T // _NW
_K = 128
_NCHUNK = _EPW // _K
_VPC = _K // _L


def _sc_body(logp_hbm, z_hbm, y_hbm, out_hbm, z_v, y_v, idx0_v, idx1_v,
             gbuf0, gbuf1, acc_v, sem0, sem1):
    wid = lax.axis_index("s") * _NC + lax.axis_index("c")
    base_e = wid * _EPW

    pltpu.sync_copy(z_hbm.at[pl.ds(base_e, _EPW)], z_v)
    pltpu.sync_copy(y_hbm.at[pl.ds(base_e, _EPW)], y_v)

    sems = (sem0, sem1)
    idxs = (idx0_v, idx1_v)
    gbufs = (gbuf0, gbuf1)

    def flat_idx(off):
        e = off + lax.iota(jnp.int32, _L)
        z = z_v[pl.ds(off, _L)]
        y = y_v[pl.ds(off, _L)]
        n_loc = lax.shift_right_logical(e * 5243, 20)
        t = e - n_loc * _T
        tt = jnp.minimum(t + z, _T - 1)
        flat = (wid * (_EPW // _T) + n_loc) * (_T * _C) + tt * _C + y
        return flat, t, z

    def fire(j, b):
        for v in range(_VPC):
            off = j * _K + v * _L
            flat, _, _ = flat_idx(off)
            idxs[b][pl.ds(v * _L, _L)] = flat
        pltpu.make_async_copy(logp_hbm.at[idxs[b]], gbufs[b],
                              sems[b]).start()

    def consume(j, b, acc):
        pltpu.make_async_copy(logp_hbm.at[idxs[b]], gbufs[b],
                              sems[b]).wait()
        for v in range(_VPC):
            off = j * _K + v * _L
            flat, t, z = flat_idx(off)
            lc = gbufs[b][pl.ds(v * _L, _L)]
            tf = t.astype(jnp.float32)
            zf = z.astype(jnp.float32)
            w = (1.0 - tf * (1.0 / _T)) * (1.0 - zf * (1.0 / _T))
            acc = acc + (-_ALPHA) * lc - (1.0 - _ALPHA) * jnp.exp(lc) * w
        return acc

    fire(0, 0)
    fire(1, 1)

    def body(j2, acc):
        j = j2 * 2
        acc = consume(j, 0, acc)
        fire(j + 2, 0)
        acc = consume(j + 1, 1, acc)
        fire(j + 3, 1)
        return acc

    acc = jnp.zeros((_L,), jnp.float32)
    acc = lax.fori_loop(0, _NCHUNK // 2 - 1, body, acc)
    acc = consume(_NCHUNK - 2, 0, acc)
    acc = consume(_NCHUNK - 1, 1, acc)

    acc_v[...] = acc
    pltpu.sync_copy(acc_v, out_hbm.at[wid])


_sc_kernel = pl.kernel(
    _sc_body,
    mesh=plsc.VectorSubcoreMesh(core_axis_name="c", subcore_axis_name="s"),
    out_type=jax.ShapeDtypeStruct((_NW, _L), jnp.float32),
    scratch_types=[
        pltpu.VMEM((_EPW,), jnp.int32),
        pltpu.VMEM((_EPW,), jnp.int32),
        pltpu.VMEM((_K,), jnp.int32),
        pltpu.VMEM((_K,), jnp.int32),
        pltpu.VMEM((_K,), jnp.float32),
        pltpu.VMEM((_K,), jnp.float32),
        pltpu.VMEM((_L,), jnp.float32),
        pltpu.SemaphoreType.DMA,
        pltpu.SemaphoreType.DMA,
    ],
)


def kernel(log_class_probabilities, timestamps_left, y_true):
    logp_flat = log_class_probabilities.reshape(-1)
    z = timestamps_left.astype(jnp.int32).reshape(-1)
    y = y_true.astype(jnp.int32).reshape(-1)
    partials = _sc_kernel(logp_flat, z, y)
    return partials.sum() * (1.0 / _N)


# speedup vs baseline: 12.3901x; 4.5959x over previous
import jax
import jax.numpy as jnp
from jax import lax
from jax.experimental import pallas as pl
from jax.experimental.pallas import tpu as pltpu
from jax.experimental.pallas import tpu_sc as plsc

_N, _T, _C = 4096, 200, 64
_ALPHA = 0.5
_L = 16
_NC, _NS = 2, 16
_NW = _NC * _NS
_EPW = _N * _T // _NW
_K = 128
_NCHUNK = _EPW // _K
_VPC = _K // _L


def _sc_body(logp_hbm, z_hbm, y_hbm, out_hbm, z_v, y_v, idx0_v, idx1_v,
             gbuf0, gbuf1, acc_v, sem0, sem1):
    wid = lax.axis_index("s") * _NC + lax.axis_index("c")
    base_e = wid * _EPW

    pltpu.sync_copy(z_hbm.at[pl.ds(base_e, _EPW)], z_v)
    pltpu.sync_copy(y_hbm.at[pl.ds(base_e, _EPW)], y_v)

    sems = (sem0, sem1)
    idxs = (idx0_v, idx1_v)
    gbufs = (gbuf0, gbuf1)

    def flat_idx(off):
        e = off + lax.iota(jnp.int32, _L)
        z = z_v[pl.ds(off, _L)]
        y = y_v[pl.ds(off, _L)]
        n_loc = lax.shift_right_logical(e * 5243, 20)
        t = e - n_loc * _T
        tt = jnp.minimum(t + z, _T - 1)
        flat = (tt * (_C * _N)
                + lax.shift_right_logical(y, 3) * (8 * _N)
                + wid * 1024
                + jnp.bitwise_and(y, 7) * 128
                + n_loc)
        return flat, t, z

    def fire(j, b):
        for v in range(_VPC):
            off = j * _K + v * _L
            flat, _, _ = flat_idx(off)
            idxs[b][pl.ds(v * _L, _L)] = flat
        pltpu.make_async_copy(logp_hbm.at[idxs[b]], gbufs[b],
                              sems[b]).start()

    def consume(j, b, acc):
        pltpu.make_async_copy(logp_hbm.at[idxs[b]], gbufs[b],
                              sems[b]).wait()
        for v in range(_VPC):
            off = j * _K + v * _L
            flat, t, z = flat_idx(off)
            lc = gbufs[b][pl.ds(v * _L, _L)]
            tf = t.astype(jnp.float32)
            zf = z.astype(jnp.float32)
            w = (1.0 - tf * (1.0 / _T)) * (1.0 - zf * (1.0 / _T))
            acc = acc + (-_ALPHA) * lc - (1.0 - _ALPHA) * jnp.exp(lc) * w
        return acc

    fire(0, 0)
    fire(1, 1)

    def body(j2, acc):
        j = j2 * 2
        acc = consume(j, 0, acc)
        fire(j + 2, 0)
        acc = consume(j + 1, 1, acc)
        fire(j + 3, 1)
        return acc

    acc = jnp.zeros((_L,), jnp.float32)
    acc = lax.fori_loop(0, _NCHUNK // 2 - 1, body, acc)
    acc = consume(_NCHUNK - 2, 0, acc)
    acc = consume(_NCHUNK - 1, 1, acc)

    acc_v[...] = acc
    pltpu.sync_copy(acc_v, out_hbm.at[wid])


_sc_kernel = pl.kernel(
    _sc_body,
    mesh=plsc.VectorSubcoreMesh(core_axis_name="c", subcore_axis_name="s"),
    out_type=jax.ShapeDtypeStruct((_NW, _L), jnp.float32),
    scratch_types=[
        pltpu.VMEM((_EPW,), jnp.int32),
        pltpu.VMEM((_EPW,), jnp.int32),
        pltpu.VMEM((_K,), jnp.int32),
        pltpu.VMEM((_K,), jnp.int32),
        pltpu.VMEM((_K,), jnp.float32),
        pltpu.VMEM((_K,), jnp.float32),
        pltpu.VMEM((_L,), jnp.float32),
        pltpu.SemaphoreType.DMA,
        pltpu.SemaphoreType.DMA,
    ],
)


def kernel(log_class_probabilities, timestamps_left, y_true):
    logp_flat = (log_class_probabilities
                 .transpose(1, 2, 0)
                 .reshape(_T, _C // 8, 8, _N // 128, 128)
                 .transpose(0, 1, 3, 2, 4)
                 .reshape(-1))
    z = timestamps_left.astype(jnp.int32).reshape(-1)
    y = y_true.astype(jnp.int32).reshape(-1)
    partials = _sc_kernel(logp_flat, z, y)
    return partials.sum() * (1.0 / _N)


# speedup vs baseline: 16.0657x; 1.2967x over previous
import jax
import jax.numpy as jnp
from jax import lax
from jax.experimental import pallas as pl
from jax.experimental.pallas import tpu as pltpu
from jax.experimental.pallas import tpu_sc as plsc

_N, _T, _C = 4096, 200, 64
_ALPHA = 0.5
_L = 16
_NC, _NS = 2, 16
_NW = _NC * _NS
_EPW = _N * _T // _NW
_K = 128
_NCHUNK = _EPW // _K
_VPC = _K // _L
_NBUF = 4


def _sc_body(logp_hbm, z_hbm, y_hbm, out_hbm, z_v, y_v, idx0, idx1, idx2,
             idx3, gb0, gb1, gb2, gb3, acc_v, sem0, sem1, sem2, sem3):
    wid = lax.axis_index("s") * _NC + lax.axis_index("c")

    for t1 in range(_T // 8):
        off = (t1 * _NW + wid) * 1024
        pltpu.sync_copy(z_hbm.at[pl.ds(off, 1024)],
                        z_v.at[pl.ds(t1 * 1024, 1024)])
        pltpu.sync_copy(y_hbm.at[pl.ds(off, 1024)],
                        y_v.at[pl.ds(t1 * 1024, 1024)])

    idxs = (idx0, idx1, idx2, idx3)
    gbufs = (gb0, gb1, gb2, gb3)
    sems = (sem0, sem1, sem2, sem3)

    def zy(j, v):
        o = j * _K + v * _L
        return z_v[pl.ds(o, _L)], y_v[pl.ds(o, _L)]

    def fire(j, b):
        for v in range(_VPC):
            z, y = zy(j, v)
            tt = jnp.minimum(j + z, _T - 1)
            flat = (tt * (_C * _N)
                    + lax.shift_right_logical(y, 3) * (8 * _N)
                    + wid * 1024
                    + jnp.bitwise_and(y, 7) * _K
                    + v * _L + lax.iota(jnp.int32, _L))
            idxs[b][pl.ds(v * _L, _L)] = flat
        pltpu.make_async_copy(logp_hbm.at[idxs[b]], gbufs[b],
                              sems[b]).start()

    def consume(j, b, acc):
        pltpu.make_async_copy(logp_hbm.at[idxs[b]], gbufs[b],
                              sems[b]).wait()
        sa = 0.5 * (1.0 - j.astype(jnp.float32) * (1.0 / _T))
        sb = sa * (1.0 / _T)
        for v in range(_VPC):
            z, _ = zy(j, v)
            lc = gbufs[b][pl.ds(v * _L, _L)]
            w = sa - sb * z.astype(jnp.float32)
            acc = acc + (-_ALPHA) * lc - jnp.exp(lc) * w
        return acc

    for b in range(_NBUF):
        fire(jnp.int32(b), b)

    def body(j4, acc):
        j = j4 * _NBUF
        for b in range(_NBUF):
            acc = consume(j + b, b, acc)
            fire(j + b + _NBUF, b)
        return acc

    acc = jnp.zeros((_L,), jnp.float32)
    acc = lax.fori_loop(0, _NCHUNK // _NBUF - 1, body, acc)
    base = jnp.int32(_NCHUNK - _NBUF)
    for b in range(_NBUF):
        acc = consume(base + b, b, acc)

    acc_v[...] = acc
    pltpu.sync_copy(acc_v, out_hbm.at[wid])


_sc_kernel = pl.kernel(
    _sc_body,
    mesh=plsc.VectorSubcoreMesh(core_axis_name="c", subcore_axis_name="s"),
    out_type=jax.ShapeDtypeStruct((_NW, _L), jnp.float32),
    scratch_types=[
        pltpu.VMEM((_EPW,), jnp.int32),
        pltpu.VMEM((_EPW,), jnp.int32),
        pltpu.VMEM((_K,), jnp.int32),
        pltpu.VMEM((_K,), jnp.int32),
        pltpu.VMEM((_K,), jnp.int32),
        pltpu.VMEM((_K,), jnp.int32),
        pltpu.VMEM((_K,), jnp.float32),
        pltpu.VMEM((_K,), jnp.float32),
        pltpu.VMEM((_K,), jnp.float32),
        pltpu.VMEM((_K,), jnp.float32),
        pltpu.VMEM((_L,), jnp.float32),
        pltpu.SemaphoreType.DMA,
        pltpu.SemaphoreType.DMA,
        pltpu.SemaphoreType.DMA,
        pltpu.SemaphoreType.DMA,
    ],
)


def _tile_flat(a):
    return (a.T.reshape(_T // 8, 8, _N // 128, 128)
            .transpose(0, 2, 1, 3)
            .reshape(-1))


def kernel(log_class_probabilities, timestamps_left, y_true):
    logp_flat = (log_class_probabilities
                 .transpose(1, 2, 0)
                 .reshape(_T, _C // 8, 8, _N // 128, 128)
                 .transpose(0, 1, 3, 2, 4)
                 .reshape(-1))
    z = _tile_flat(timestamps_left.astype(jnp.int32))
    y = _tile_flat(y_true.astype(jnp.int32))
    partials = _sc_kernel(logp_flat, z, y)
    return partials.sum() * (1.0 / _N)


# speedup vs baseline: 26.5616x; 1.6533x over previous
import jax
import jax.numpy as jnp
from jax import lax
from jax.experimental import pallas as pl
from jax.experimental.pallas import tpu as pltpu
from jax.experimental.pallas import tpu_sc as plsc

_N, _T, _C = 4096, 200, 64
_ALPHA = 0.5
_L = 16
_NC, _NS = 2, 16
_NW = _NC * _NS
_EPW = _N * _T // _NW
_K = 128
_NCHUNK = _EPW // _K
_VPC = _K // _L
_NBUF = 8


def _sc_body(logp_hbm, z_hbm, y_hbm, out_hbm, z_v, y_v, idx0, idx1, idx2,
             idx3, idx4, idx5, idx6, idx7, gb0, gb1, gb2, gb3, gb4, gb5,
             gb6, gb7, acc_v, sem0, sem1, sem2, sem3, sem4, sem5, sem6,
             sem7):
    wid = lax.axis_index("s") * _NC + lax.axis_index("c")

    zy_copies = []
    for t1 in range(_T // 8):
        off = (t1 * _NW + wid) * 1024
        dst = pl.ds(t1 * 1024, 1024)
        for src_hbm, dst_v in ((z_hbm, z_v), (y_hbm, y_v)):
            c = pltpu.make_async_copy(src_hbm.at[pl.ds(off, 1024)],
                                      dst_v.at[dst], sem0)
            c.start()
            zy_copies.append(c)
    for c in zy_copies:
        c.wait()

    idxs = (idx0, idx1, idx2, idx3, idx4, idx5, idx6, idx7)
    gbufs = (gb0, gb1, gb2, gb3, gb4, gb5, gb6, gb7)
    sems = (sem0, sem1, sem2, sem3, sem4, sem5, sem6, sem7)

    def zy(j, v):
        o = j * _K + v * _L
        return z_v[pl.ds(o, _L)], y_v[pl.ds(o, _L)]

    def fire(j, b):
        for v in range(_VPC):
            z, y = zy(j, v)
            tt = jnp.minimum(j + z, _T - 1)
            flat = (tt * (_C * _N)
                    + lax.shift_right_logical(y, 3) * (8 * _N)
                    + wid * 1024
                    + jnp.bitwise_and(y, 7) * _K
                    + v * _L + lax.iota(jnp.int32, _L))
            idxs[b][pl.ds(v * _L, _L)] = flat
        pltpu.make_async_copy(logp_hbm.at[idxs[b]], gbufs[b],
                              sems[b]).start()

    def consume(j, b, acc):
        pltpu.make_async_copy(logp_hbm.at[idxs[b]], gbufs[b],
                              sems[b]).wait()
        sa = 0.5 * (1.0 - j.astype(jnp.float32) * (1.0 / _T))
        sb = sa * (1.0 / _T)
        for v in range(_VPC):
            z, _ = zy(j, v)
            lc = gbufs[b][pl.ds(v * _L, _L)]
            w = sa - sb * z.astype(jnp.float32)
            acc = acc + (-_ALPHA) * lc - jnp.exp(lc) * w
        return acc

    for b in range(_NBUF):
        fire(jnp.int32(b), b)

    def body(j4, acc):
        j = j4 * _NBUF
        for b in range(_NBUF):
            acc = consume(j + b, b, acc)
            fire(j + b + _NBUF, b)
        return acc

    acc = jnp.zeros((_L,), jnp.float32)
    acc = lax.fori_loop(0, _NCHUNK // _NBUF - 1, body, acc)
    base = jnp.int32(_NCHUNK - _NBUF)
    for b in range(_NBUF):
        acc = consume(base + b, b, acc)

    acc_v[...] = acc
    pltpu.sync_copy(acc_v, out_hbm.at[wid])


_sc_kernel = pl.kernel(
    _sc_body,
    mesh=plsc.VectorSubcoreMesh(core_axis_name="c", subcore_axis_name="s"),
    out_type=jax.ShapeDtypeStruct((_NW, _L), jnp.float32),
    scratch_types=[
        pltpu.VMEM((_EPW,), jnp.int32),
        pltpu.VMEM((_EPW,), jnp.int32),
        pltpu.VMEM((_K,), jnp.int32),
        pltpu.VMEM((_K,), jnp.int32),
        pltpu.VMEM((_K,), jnp.int32),
        pltpu.VMEM((_K,), jnp.int32),
        pltpu.VMEM((_K,), jnp.int32),
        pltpu.VMEM((_K,), jnp.int32),
        pltpu.VMEM((_K,), jnp.int32),
        pltpu.VMEM((_K,), jnp.int32),
        pltpu.VMEM((_K,), jnp.float32),
        pltpu.VMEM((_K,), jnp.float32),
        pltpu.VMEM((_K,), jnp.float32),
        pltpu.VMEM((_K,), jnp.float32),
        pltpu.VMEM((_K,), jnp.float32),
        pltpu.VMEM((_K,), jnp.float32),
        pltpu.VMEM((_K,), jnp.float32),
        pltpu.VMEM((_K,), jnp.float32),
        pltpu.VMEM((_L,), jnp.float32),
        pltpu.SemaphoreType.DMA,
        pltpu.SemaphoreType.DMA,
        pltpu.SemaphoreType.DMA,
        pltpu.SemaphoreType.DMA,
        pltpu.SemaphoreType.DMA,
        pltpu.SemaphoreType.DMA,
        pltpu.SemaphoreType.DMA,
        pltpu.SemaphoreType.DMA,
    ],
)


def _tile_flat(a):
    return (a.T.reshape(_T // 8, 8, _N // 128, 128)
            .transpose(0, 2, 1, 3)
            .reshape(-1))


def kernel(log_class_probabilities, timestamps_left, y_true):
    logp_flat = (log_class_probabilities
                 .transpose(1, 2, 0)
                 .reshape(_T, _C // 8, 8, _N // 128, 128)
                 .transpose(0, 1, 3, 2, 4)
                 .reshape(-1))
    z = _tile_flat(timestamps_left.astype(jnp.int32))
    y = _tile_flat(y_true.astype(jnp.int32))
    partials = _sc_kernel(logp_flat, z, y)
    return partials.sum() * (1.0 / _N)
